# Initial kernel scaffold; baseline (speedup 1.0000x reference)
import jax
import jax.numpy as jnp
from jax.experimental import pallas as pl


def kernel(hidden_states, Wq, bq, Wk, bk, Wv, bv, Wao, bao, ln1_g, ln1_b, Wr, br, We, be, Wo, bo, ln2_g, ln2_b):
    raise NotImplementedError("write your pallas kernel here")


# speedup vs baseline: 1.7728x; 1.7728x over previous
import functools

import jax
import jax.numpy as jnp
from jax.experimental import pallas as pl
from jax.experimental.pallas import tpu as pltpu

S, D, H, DFF, E = 2048, 1024, 16, 4096, 8
DH = D // H
TQ = 1024
TKB = 1024
TR = 256
T = 256
MAXP = S + E * T
MAX_TILES = MAXP // T
DFFC = 2048
NDC = DFF // DFFC
F32 = jnp.float32
BF16 = jnp.bfloat16
HIGHEST = jax.lax.Precision.HIGHEST


_NN = (((1,), (0,)), ((), ()))
_NT = (((1,), (1,)), ((), ()))


def _hi_lo(x):
    h = x.astype(BF16)
    l = (x - h.astype(F32)).astype(BF16)
    return h, l


def _dot1(x, w, dn=_NN):
    return jax.lax.dot_general(x.astype(BF16), w.astype(BF16),
                               dimension_numbers=dn,
                               preferred_element_type=F32)


def _split3(x):
    h = x.astype(BF16)
    r = x - h.astype(F32)
    m = r.astype(BF16)
    l = (r - m.astype(F32)).astype(BF16)
    return h, m, l


def _dot6(x, w, dn=_NN):
    xh, xm, xl = _split3(x)
    wh, wm, wl = _split3(w)
    d = functools.partial(jax.lax.dot_general, dimension_numbers=dn,
                          preferred_element_type=F32)
    small = d(xh, wl) + d(xl, wh) + d(xm, wm)
    mid = d(xh, wm) + d(xm, wh)
    return (small + mid) + d(xh, wh)


def _qkv_body(x_ref, w_ref, b_ref, o_ref):
    o_ref[...] = _dot1(x_ref[...], w_ref[...]) + b_ref[...]


def _attn_body(q_ref, k_ref, v_ref, o_ref, m_ref, l_ref):
    kb = pl.program_id(2)

    @pl.when(kb == 0)
    def _():
        o_ref[0] = jnp.zeros_like(o_ref[0])
        m_ref[...] = jnp.full_like(m_ref[...], -jnp.inf)
        l_ref[...] = jnp.zeros_like(l_ref[...])

    s = _dot1(q_ref[0], k_ref[0], _NT) * 0.125
    m_blk = jnp.max(s, axis=-1, keepdims=True)
    m_old = m_ref[...]
    m_new = jnp.maximum(m_old, m_blk)
    corr = jnp.where(m_old == m_new, jnp.float32(0.0), m_old - m_new)
    u = jnp.exp(s - m_new)
    l_blk = jnp.sum(u, axis=-1, keepdims=True)
    l_old = l_ref[...]
    ecorr = jnp.exp(corr)
    l_new = ecorr * l_old + l_blk
    acc = (ecorr * l_old) * o_ref[0]
    res = acc + _dot1(u, v_ref[0])
    o_ref[0] = res * (1.0 / l_new)
    m_ref[...] = m_new
    l_ref[...] = l_new


def _proj_body(ctx_ref, wao_ref, bao_ref, x_ref, g_ref, b_ref, wr_ref, br_ref,
               ao_ref, sc_ref, eid_ref):
    t = _dot1(ctx_ref[...], wao_ref[...]) + bao_ref[...] + x_ref[...]
    mu = jnp.mean(t, axis=-1, keepdims=True)
    var = jnp.mean((t - mu) ** 2, axis=-1, keepdims=True)
    ao = (t - mu) / jnp.sqrt(var + 1e-12) * g_ref[...] + b_ref[...]
    ao_ref[...] = ao
    logits = _dot1(ao, wr_ref[...]) + br_ref[...]
    lane = jax.lax.broadcasted_iota(jnp.int32, logits.shape, 1)
    logits = jnp.where(lane < E, logits, jnp.float32(-1e30))
    m = jnp.max(logits, axis=-1, keepdims=True)
    p = jnp.exp(logits - m)
    p = p / jnp.sum(p, axis=-1, keepdims=True)
    sc_ref[...] = p[:, :E]
    mp = jnp.max(p, axis=-1, keepdims=True)
    eid_ref[...] = jnp.min(jnp.where(p == mp, lane, E), axis=-1,
                           keepdims=True)


def _moe_body(te_ref, act_ref, x_ref, we_ref, be_ref, wo_ref, y_ref):
    t = pl.program_id(0)
    c = pl.program_id(1)

    @pl.when(act_ref[t] != 0)
    def _():
        xb = x_ref[...].astype(BF16)
        we = we_ref[0].astype(BF16)
        h = jax.lax.dot_general(xb, we, (((1,), (1,)), ((), ())),
                                preferred_element_type=F32)
        h = h + be_ref[0]
        h = 0.5 * h * (1.0 + jax.lax.erf(h * 0.7071067811865476))
        y = jax.lax.dot_general(h.astype(BF16), wo_ref[...],
                                (((1,), (1,)), ((), ())),
                                preferred_element_type=F32)

        @pl.when(c == 0)
        def _():
            y_ref[...] = y

        @pl.when(c != 0)
        def _():
            y_ref[...] += y


def _ln2_body(y_ref, bo_ref, ao_ref, g_ref, b_ref, o_ref):
    tot = y_ref[...] + bo_ref[...] + ao_ref[...]
    mu = jnp.mean(tot, axis=-1, keepdims=True)
    var = jnp.mean((tot - mu) ** 2, axis=-1, keepdims=True)
    o_ref[...] = (tot - mu) / jnp.sqrt(var + 1e-12) * g_ref[...] + b_ref[...]


def kernel(hidden_states, Wq, bq, Wk, bk, Wv, bv, Wao, bao, ln1_g, ln1_b,
           Wr, br, We, be, Wo, bo, ln2_g, ln2_b):
    x = hidden_states.reshape(S, D)
    wqkv = jnp.concatenate([Wq.T, Wk.T, Wv.T], axis=1)
    bqkv = jnp.concatenate([bq, bk, bv]).reshape(1, 3 * D)

    qkv = pl.pallas_call(
        _qkv_body,
        grid=(S // TR,),
        in_specs=[pl.BlockSpec((TR, D), lambda i: (i, 0)),
                  pl.BlockSpec((D, 3 * D), lambda i: (0, 0)),
                  pl.BlockSpec((1, 3 * D), lambda i: (0, 0))],
        out_specs=pl.BlockSpec((TR, 3 * D), lambda i: (i, 0)),
        out_shape=jax.ShapeDtypeStruct((S, 3 * D), F32),
    )(x, wqkv, bqkv)

    q = qkv[:, :D].reshape(S, H, DH).transpose(1, 0, 2)
    k = qkv[:, D:2 * D].reshape(S, H, DH).transpose(1, 0, 2)
    v = qkv[:, 2 * D:].reshape(S, H, DH).transpose(1, 0, 2)

    ctx3 = pl.pallas_call(
        _attn_body,
        grid=(H, S // TQ, S // TKB),
        in_specs=[pl.BlockSpec((1, TQ, DH), lambda h, i, j: (h, i, 0)),
                  pl.BlockSpec((1, TKB, DH), lambda h, i, j: (h, j, 0)),
                  pl.BlockSpec((1, TKB, DH), lambda h, i, j: (h, j, 0))],
        out_specs=pl.BlockSpec((1, TQ, DH), lambda h, i, j: (h, i, 0)),
        out_shape=jax.ShapeDtypeStruct((H, S, DH), F32),
        scratch_shapes=[pltpu.VMEM((TQ, 1), F32),
                        pltpu.VMEM((TQ, 1), F32)],
    )(q, k, v)
    ctx = ctx3.transpose(1, 0, 2).reshape(S, D)

    wr_pad = jnp.zeros((D, 128), F32).at[:, :E].set(Wr.T)
    br_pad = jnp.zeros((1, 128), F32).at[0, :E].set(br)
    ao, scores, eids = pl.pallas_call(
        _proj_body,
        grid=(S // TR,),
        in_specs=[pl.BlockSpec((TR, D), lambda i: (i, 0)),
                  pl.BlockSpec((D, D), lambda i: (0, 0)),
                  pl.BlockSpec((1, D), lambda i: (0, 0)),
                  pl.BlockSpec((TR, D), lambda i: (i, 0)),
                  pl.BlockSpec((1, D), lambda i: (0, 0)),
                  pl.BlockSpec((1, D), lambda i: (0, 0)),
                  pl.BlockSpec((D, 128), lambda i: (0, 0)),
                  pl.BlockSpec((1, 128), lambda i: (0, 0))],
        out_specs=[pl.BlockSpec((TR, D), lambda i: (i, 0)),
                   pl.BlockSpec((TR, E), lambda i: (i, 0)),
                   pl.BlockSpec((TR, 1), lambda i: (i, 0))],
        out_shape=[jax.ShapeDtypeStruct((S, D), F32),
                   jax.ShapeDtypeStruct((S, E), F32),
                   jax.ShapeDtypeStruct((S, 1), jnp.int32)],
    )(ctx, Wao.T, bao.reshape(1, D), x, ln1_g.reshape(1, D),
      ln1_b.reshape(1, D), wr_pad, br_pad)

    eid = eids[:, 0]
    order = jnp.argsort(eid, stable=True)
    counts = jnp.sum(eid[:, None] == jnp.arange(E, dtype=jnp.int32)[None, :],
                     axis=0).astype(jnp.int32)
    padded = ((counts + T - 1) // T) * T
    zero1 = jnp.zeros((1,), jnp.int32)
    base = jnp.concatenate([zero1, jnp.cumsum(padded)])
    starts = jnp.concatenate([zero1, jnp.cumsum(counts)])
    se = eid[order]
    pos_sorted = base[se] + jnp.arange(S, dtype=jnp.int32) - starts[se]
    x_pad = jnp.zeros((MAXP, D), F32).at[pos_sorted].set(ao[order])
    pos = jnp.zeros((S,), jnp.int32).at[order].set(pos_sorted)
    ntiles = base[E] // T
    tidx = jnp.arange(MAX_TILES, dtype=jnp.int32)
    te = jnp.sum(tidx[:, None] >= (base[1:] // T)[None, :], axis=1)
    te = jnp.minimum(te, E - 1).astype(jnp.int32)
    act = (tidx < ntiles).astype(jnp.int32)
    last_e = te[jnp.maximum(ntiles - 1, 0)]
    te = jnp.where(act == 1, te, last_e).astype(jnp.int32)

    grid_spec = pltpu.PrefetchScalarGridSpec(
        num_scalar_prefetch=2,
        grid=(MAX_TILES, NDC),
        in_specs=[pl.BlockSpec((T, D), lambda t, c, te, a: (t, 0)),
                  pl.BlockSpec((1, DFFC, D), lambda t, c, te, a: (te[t], c, 0)),
                  pl.BlockSpec((1, 1, DFFC),
                               lambda t, c, te, a: (te[t] * NDC + c, 0, 0)),
                  pl.BlockSpec((D, DFFC), lambda t, c, te, a: (0, c))],
        out_specs=pl.BlockSpec((T, D), lambda t, c, te, a: (t, 0)),
    )
    y_pad = pl.pallas_call(
        _moe_body,
        grid_spec=grid_spec,
        out_shape=jax.ShapeDtypeStruct((MAXP, D), F32),
    )(te, act, x_pad, We, be.reshape(E * NDC, 1, DFFC), Wo.astype(BF16), )

    y = y_pad[pos]

    out = pl.pallas_call(
        _ln2_body,
        grid=(S // TR,),
        in_specs=[pl.BlockSpec((TR, D), lambda i: (i, 0)),
                  pl.BlockSpec((1, D), lambda i: (0, 0)),
                  pl.BlockSpec((TR, D), lambda i: (i, 0)),
                  pl.BlockSpec((1, D), lambda i: (0, 0)),
                  pl.BlockSpec((1, D), lambda i: (0, 0))],
        out_specs=pl.BlockSpec((TR, D), lambda i: (i, 0)),
        out_shape=jax.ShapeDtypeStruct((S, D), F32),
    )(y, bo.reshape(1, D), ao, ln2_g.reshape(1, D), ln2_b.reshape(1, D))

    return (out.reshape(1, S, D), scores.reshape(1, S, E))


# speedup vs baseline: 1.9217x; 1.0840x over previous
import functools

import jax
import jax.numpy as jnp
from jax import lax
from jax.experimental import pallas as pl
from jax.experimental.pallas import tpu as pltpu
from jax.experimental.pallas import tpu_sc as plsc

S, D, H, DFF, E = 2048, 1024, 16, 4096, 8
DH = D // H
TQ = 1024
TKB = 1024
TR = 256
T = 256
MAXP = S + E * T
MAX_TILES = MAXP // T
DFFC = 2048
NDC = DFF // DFFC
F32 = jnp.float32
BF16 = jnp.bfloat16
HIGHEST = jax.lax.Precision.HIGHEST


_NN = (((1,), (0,)), ((), ()))
_NT = (((1,), (1,)), ((), ()))


def _hi_lo(x):
    h = x.astype(BF16)
    l = (x - h.astype(F32)).astype(BF16)
    return h, l


def _dot1(x, w, dn=_NN):
    return jax.lax.dot_general(x.astype(BF16), w.astype(BF16),
                               dimension_numbers=dn,
                               preferred_element_type=F32)


def _split3(x):
    h = x.astype(BF16)
    r = x - h.astype(F32)
    m = r.astype(BF16)
    l = (r - m.astype(F32)).astype(BF16)
    return h, m, l


def _dot6(x, w, dn=_NN):
    xh, xm, xl = _split3(x)
    wh, wm, wl = _split3(w)
    d = functools.partial(jax.lax.dot_general, dimension_numbers=dn,
                          preferred_element_type=F32)
    small = d(xh, wl) + d(xl, wh) + d(xm, wm)
    mid = d(xh, wm) + d(xm, wh)
    return (small + mid) + d(xh, wh)


def _qkv_body(x_ref, w_ref, b_ref, o_ref):
    o_ref[...] = _dot1(x_ref[...], w_ref[...]) + b_ref[...]


def _attn_body(q_ref, k_ref, v_ref, o_ref, m_ref, l_ref):
    kb = pl.program_id(2)

    @pl.when(kb == 0)
    def _():
        o_ref[0] = jnp.zeros_like(o_ref[0])
        m_ref[...] = jnp.full_like(m_ref[...], -jnp.inf)
        l_ref[...] = jnp.zeros_like(l_ref[...])

    s = _dot1(q_ref[0], k_ref[0], _NT) * 0.125
    m_blk = jnp.max(s, axis=-1, keepdims=True)
    m_old = m_ref[...]
    m_new = jnp.maximum(m_old, m_blk)
    corr = jnp.where(m_old == m_new, jnp.float32(0.0), m_old - m_new)
    u = jnp.exp(s - m_new)
    l_blk = jnp.sum(u, axis=-1, keepdims=True)
    l_old = l_ref[...]
    ecorr = jnp.exp(corr)
    l_new = ecorr * l_old + l_blk
    acc = (ecorr * l_old) * o_ref[0]
    res = acc + _dot1(u, v_ref[0])
    o_ref[0] = res * (1.0 / l_new)
    m_ref[...] = m_new
    l_ref[...] = l_new


def _proj_body(ctx_ref, wao_ref, bao_ref, x_ref, g_ref, b_ref, wr_ref, br_ref,
               ao_ref, sc_ref, eid_ref):
    t = _dot1(ctx_ref[...], wao_ref[...]) + bao_ref[...] + x_ref[...]
    mu = jnp.mean(t, axis=-1, keepdims=True)
    var = jnp.mean((t - mu) ** 2, axis=-1, keepdims=True)
    ao = (t - mu) / jnp.sqrt(var + 1e-12) * g_ref[...] + b_ref[...]
    ao_ref[...] = ao
    logits = _dot1(ao, wr_ref[...]) + br_ref[...]
    lane = jax.lax.broadcasted_iota(jnp.int32, logits.shape, 1)
    logits = jnp.where(lane < E, logits, jnp.float32(-1e30))
    m = jnp.max(logits, axis=-1, keepdims=True)
    p = jnp.exp(logits - m)
    p = p / jnp.sum(p, axis=-1, keepdims=True)
    sc_ref[...] = p[:, :E]
    mp = jnp.max(p, axis=-1, keepdims=True)
    eid_ref[...] = jnp.min(jnp.where(p == mp, lane, E), axis=-1,
                           keepdims=True)


def _moe_body(te_ref, act_ref, x_ref, we_ref, be_ref, wo_ref, y_ref):
    t = pl.program_id(0)
    c = pl.program_id(1)

    @pl.when(act_ref[t] != 0)
    def _():
        xb = x_ref[...].astype(BF16)
        we = we_ref[0].astype(BF16)
        h = jax.lax.dot_general(xb, we, (((1,), (1,)), ((), ())),
                                preferred_element_type=F32)
        h = h + be_ref[0]
        h = 0.5 * h * (1.0 + jax.lax.erf(h * 0.7071067811865476))
        y = jax.lax.dot_general(h.astype(BF16), wo_ref[...],
                                (((1,), (1,)), ((), ())),
                                preferred_element_type=F32)

        @pl.when(c == 0)
        def _():
            y_ref[...] = y

        @pl.when(c != 0)
        def _():
            y_ref[...] += y


NSUB = 16
CHUNK = S // NSUB
HALF = CHUNK // 2
NGRP = CHUNK // 16
LOG2T = 8

_sc_mesh = plsc.VectorSubcoreMesh(core_axis_name="c", subcore_axis_name="s")


@functools.partial(
    pl.kernel,
    mesh=_sc_mesh,
    out_type=[
        jax.ShapeDtypeStruct((MAXP, D), jnp.float32),
        jax.ShapeDtypeStruct((S,), jnp.int32),
        jax.ShapeDtypeStruct((MAX_TILES,), jnp.int32),
        jax.ShapeDtypeStruct((MAX_TILES,), jnp.int32),
    ],
    scratch_types=[
        pltpu.VMEM((S,), jnp.int32),
        pltpu.VMEM((2, HALF), jnp.int32),
        pltpu.VMEM((HALF, D), jnp.float32),
        pltpu.VMEM((16,), jnp.int32),
        pltpu.SemaphoreType.DMA,
    ],
)
def _sc_dispatch(eids_hbm, ao_hbm, x_pad_hbm, pos_hbm, te_hbm, act_hbm,
                 eids_v, pos_v, rows_v, meta_v, sem):
    c = lax.axis_index("c")
    s = lax.axis_index("s")
    lane = lax.iota(jnp.int32, 16)
    one = jnp.full((16,), 1, jnp.int32)
    zero = jnp.zeros((16,), jnp.int32)

    def take16(x, idx):
        return x.at[idx].get(mode="promise_in_bounds")

    def bcast(x, j):
        return take16(x, jnp.full((16,), j, jnp.int32))

    def eq_mask(a, b):
        return one - jnp.minimum(jnp.abs(a - b), one)

    def ge_mask(a, b):
        return jnp.minimum(jnp.maximum(a - b + 1, zero), one)

    def sum16(x):
        for kk in (1, 2, 4, 8):
            x = x + take16(x, (lane + kk) & 15)
        return x

    def cumsum16(x):
        for kk in (1, 2, 4, 8):
            sh = take16(x, jnp.maximum(lane - kk, 0))
            x = x + ge_mask(lane, jnp.full((16,), kk, jnp.int32)) * sh
        return x

    pltpu.sync_copy(eids_hbm, eids_v)
    e_consts = [jnp.full((16,), e, jnp.int32) for e in range(E)]
    lane_e = [eq_mask(lane, e_consts[e]) for e in range(E)]
    sgrp = jnp.broadcast_to(s * NGRP, (16,)).astype(jnp.int32)
    acc_t = [zero for _ in range(E)]
    acc_p = [zero for _ in range(E)]
    for gi in range(NSUB * NGRP):
        ev = eids_v[pl.ds(gi * 16, 16)]
        infl = ge_mask(sgrp, jnp.full((16,), gi + 1, jnp.int32))
        for e in range(E):
            m = eq_mask(ev, e_consts[e])
            acc_t[e] = acc_t[e] + m
            acc_p[e] = acc_p[e] + infl * m
    tot = zero
    prefix = zero
    for e in range(E):
        tot = tot + lane_e[e] * sum16(acc_t[e])
        prefix = prefix + lane_e[e] * sum16(acc_p[e])
    padded = ((tot + (T - 1)) >> LOG2T) << LOG2T
    incl = cumsum16(padded)
    base = incl - padded
    ntiles_v = bcast(incl, 15) >> LOG2T
    off = base + prefix

    for g in range(NGRP):
        ev = eids_v[pl.ds(s * CHUNK + g * 16, 16)]
        pv = jnp.zeros((16,), jnp.int32)
        for e in range(E):
            e_vec = jnp.full((16,), e, jnp.int32)
            m = eq_mask(ev, e_vec)
            csum = cumsum16(m)
            rank = csum - 1
            off_e = bcast(off, e)
            pv = m * (off_e + rank) + (one - m) * pv
            off = off + eq_mask(lane, e_vec) * bcast(csum, 15)
        pos_v[g // (NGRP // 2), pl.ds((g % (NGRP // 2)) * 16, 16)] = pv

    pltpu.sync_copy(pos_v.at[c], pos_hbm.at[pl.ds(s * CHUNK + c * HALF, HALF)])
    pltpu.sync_copy(ao_hbm.at[pl.ds(s * CHUNK + c * HALF, HALF)], rows_v)
    pltpu.async_copy(rows_v, x_pad_hbm.at[pos_v.at[c]], sem).wait()

    @pl.when(jnp.logical_and(c == 0, s == 0))
    def _():
        te = jnp.zeros((16,), jnp.int32)
        endt = (base + padded) >> LOG2T
        for e in range(E):
            te = te + ge_mask(lane, bcast(endt, e))
        te = jnp.minimum(te, E - 1)
        last_e = take16(te, jnp.maximum(ntiles_v - 1, zero))
        act = one - ge_mask(lane, ntiles_v)
        te = act * te + (one - act) * last_e
        meta_v[...] = te
        pltpu.sync_copy(meta_v, te_hbm)
        meta_v[...] = act
        pltpu.sync_copy(meta_v, act_hbm)


@functools.partial(
    pl.kernel,
    mesh=_sc_mesh,
    out_type=jax.ShapeDtypeStruct((S, D), jnp.float32),
    scratch_types=[
        pltpu.VMEM((2, HALF), jnp.int32),
        pltpu.VMEM((HALF, D), jnp.float32),
        pltpu.SemaphoreType.DMA,
    ],
)
def _sc_unsort(pos_hbm, y_pad_hbm, y_hbm, pos_v, rows_v, sem):
    c = lax.axis_index("c")
    s = lax.axis_index("s")
    base = s * CHUNK + c * HALF
    pltpu.sync_copy(pos_hbm.at[pl.ds(base, HALF)], pos_v.at[c])
    pltpu.async_copy(y_pad_hbm.at[pos_v.at[c]], rows_v, sem).wait()
    pltpu.sync_copy(rows_v, y_hbm.at[pl.ds(base, HALF)])


def _ln2_body(y_ref, bo_ref, ao_ref, g_ref, b_ref, o_ref):
    tot = y_ref[...] + bo_ref[...] + ao_ref[...]
    mu = jnp.mean(tot, axis=-1, keepdims=True)
    var = jnp.mean((tot - mu) ** 2, axis=-1, keepdims=True)
    o_ref[...] = (tot - mu) / jnp.sqrt(var + 1e-12) * g_ref[...] + b_ref[...]


def kernel(hidden_states, Wq, bq, Wk, bk, Wv, bv, Wao, bao, ln1_g, ln1_b,
           Wr, br, We, be, Wo, bo, ln2_g, ln2_b):
    x = hidden_states.reshape(S, D)
    wqkv = jnp.concatenate([Wq.T, Wk.T, Wv.T], axis=1)
    bqkv = jnp.concatenate([bq, bk, bv]).reshape(1, 3 * D)

    qkv = pl.pallas_call(
        _qkv_body,
        grid=(S // TR,),
        in_specs=[pl.BlockSpec((TR, D), lambda i: (i, 0)),
                  pl.BlockSpec((D, 3 * D), lambda i: (0, 0)),
                  pl.BlockSpec((1, 3 * D), lambda i: (0, 0))],
        out_specs=pl.BlockSpec((TR, 3 * D), lambda i: (i, 0)),
        out_shape=jax.ShapeDtypeStruct((S, 3 * D), F32),
    )(x, wqkv, bqkv)

    q = qkv[:, :D].reshape(S, H, DH).transpose(1, 0, 2)
    k = qkv[:, D:2 * D].reshape(S, H, DH).transpose(1, 0, 2)
    v = qkv[:, 2 * D:].reshape(S, H, DH).transpose(1, 0, 2)

    ctx3 = pl.pallas_call(
        _attn_body,
        grid=(H, S // TQ, S // TKB),
        in_specs=[pl.BlockSpec((1, TQ, DH), lambda h, i, j: (h, i, 0)),
                  pl.BlockSpec((1, TKB, DH), lambda h, i, j: (h, j, 0)),
                  pl.BlockSpec((1, TKB, DH), lambda h, i, j: (h, j, 0))],
        out_specs=pl.BlockSpec((1, TQ, DH), lambda h, i, j: (h, i, 0)),
        out_shape=jax.ShapeDtypeStruct((H, S, DH), F32),
        scratch_shapes=[pltpu.VMEM((TQ, 1), F32),
                        pltpu.VMEM((TQ, 1), F32)],
    )(q, k, v)
    ctx = ctx3.transpose(1, 0, 2).reshape(S, D)

    wr_pad = jnp.zeros((D, 128), F32).at[:, :E].set(Wr.T)
    br_pad = jnp.zeros((1, 128), F32).at[0, :E].set(br)
    ao, scores, eids = pl.pallas_call(
        _proj_body,
        grid=(S // TR,),
        in_specs=[pl.BlockSpec((TR, D), lambda i: (i, 0)),
                  pl.BlockSpec((D, D), lambda i: (0, 0)),
                  pl.BlockSpec((1, D), lambda i: (0, 0)),
                  pl.BlockSpec((TR, D), lambda i: (i, 0)),
                  pl.BlockSpec((1, D), lambda i: (0, 0)),
                  pl.BlockSpec((1, D), lambda i: (0, 0)),
                  pl.BlockSpec((D, 128), lambda i: (0, 0)),
                  pl.BlockSpec((1, 128), lambda i: (0, 0))],
        out_specs=[pl.BlockSpec((TR, D), lambda i: (i, 0)),
                   pl.BlockSpec((TR, E), lambda i: (i, 0)),
                   pl.BlockSpec((TR, 1), lambda i: (i, 0))],
        out_shape=[jax.ShapeDtypeStruct((S, D), F32),
                   jax.ShapeDtypeStruct((S, E), F32),
                   jax.ShapeDtypeStruct((S, 1), jnp.int32)],
    )(ctx, Wao.T, bao.reshape(1, D), x, ln1_g.reshape(1, D),
      ln1_b.reshape(1, D), wr_pad, br_pad)

    x_pad, pos, te, act = _sc_dispatch(eids.reshape(S), ao)

    grid_spec = pltpu.PrefetchScalarGridSpec(
        num_scalar_prefetch=2,
        grid=(MAX_TILES, NDC),
        in_specs=[pl.BlockSpec((T, D), lambda t, c, te, a: (t, 0)),
                  pl.BlockSpec((1, DFFC, D), lambda t, c, te, a: (te[t], c, 0)),
                  pl.BlockSpec((1, 1, DFFC),
                               lambda t, c, te, a: (te[t] * NDC + c, 0, 0)),
                  pl.BlockSpec((D, DFFC), lambda t, c, te, a: (0, c))],
        out_specs=pl.BlockSpec((T, D), lambda t, c, te, a: (t, 0)),
    )
    y_pad = pl.pallas_call(
        _moe_body,
        grid_spec=grid_spec,
        out_shape=jax.ShapeDtypeStruct((MAXP, D), F32),
    )(te, act, x_pad, We, be.reshape(E * NDC, 1, DFFC), Wo.astype(BF16), )

    y = _sc_unsort(pos, y_pad)

    out = pl.pallas_call(
        _ln2_body,
        grid=(S // TR,),
        in_specs=[pl.BlockSpec((TR, D), lambda i: (i, 0)),
                  pl.BlockSpec((1, D), lambda i: (0, 0)),
                  pl.BlockSpec((TR, D), lambda i: (i, 0)),
                  pl.BlockSpec((1, D), lambda i: (0, 0)),
                  pl.BlockSpec((1, D), lambda i: (0, 0))],
        out_specs=pl.BlockSpec((TR, D), lambda i: (i, 0)),
        out_shape=jax.ShapeDtypeStruct((S, D), F32),
    )(y, bo.reshape(1, D), ao, ln2_g.reshape(1, D), ln2_b.reshape(1, D))

    return (out.reshape(1, S, D), scores.reshape(1, S, E))


# speedup vs baseline: 1.9992x; 1.0403x over previous
import functools

import jax
import jax.numpy as jnp
from jax import lax
from jax.experimental import pallas as pl
from jax.experimental.pallas import tpu as pltpu
from jax.experimental.pallas import tpu_sc as plsc

S, D, H, DFF, E = 2048, 1024, 16, 4096, 8
DH = D // H
TQ = 1024
TKB = 1024
TR = 256
T = 256
MAXP = S + E * T
MAX_TILES = MAXP // T
DFFC = 2048
NDC = DFF // DFFC
F32 = jnp.float32
BF16 = jnp.bfloat16
HIGHEST = jax.lax.Precision.HIGHEST


_NN = (((1,), (0,)), ((), ()))
_NT = (((1,), (1,)), ((), ()))


def _hi_lo(x):
    h = x.astype(BF16)
    l = (x - h.astype(F32)).astype(BF16)
    return h, l


def _dot1(x, w, dn=_NN):
    return jax.lax.dot_general(x.astype(BF16), w.astype(BF16),
                               dimension_numbers=dn,
                               preferred_element_type=F32)


def _split3(x):
    h = x.astype(BF16)
    r = x - h.astype(F32)
    m = r.astype(BF16)
    l = (r - m.astype(F32)).astype(BF16)
    return h, m, l


def _dot6(x, w, dn=_NN):
    xh, xm, xl = _split3(x)
    wh, wm, wl = _split3(w)
    d = functools.partial(jax.lax.dot_general, dimension_numbers=dn,
                          preferred_element_type=F32)
    small = d(xh, wl) + d(xl, wh) + d(xm, wm)
    mid = d(xh, wm) + d(xm, wh)
    return (small + mid) + d(xh, wh)


def _qkv_body(x_ref, w_ref, b_ref, o_ref):
    o_ref[...] = _dot1(x_ref[...], w_ref[...]) + b_ref[...]


def _attn_body(q_ref, k_ref, v_ref, o_ref, m_ref, l_ref):
    kb = pl.program_id(2)

    @pl.when(kb == 0)
    def _():
        o_ref[0] = jnp.zeros_like(o_ref[0])
        m_ref[...] = jnp.full_like(m_ref[...], -jnp.inf)
        l_ref[...] = jnp.zeros_like(l_ref[...])

    s = _dot1(q_ref[0], k_ref[0], _NT) * 0.125
    m_blk = jnp.max(s, axis=-1, keepdims=True)
    m_old = m_ref[...]
    m_new = jnp.maximum(m_old, m_blk)
    corr = jnp.where(m_old == m_new, jnp.float32(0.0), m_old - m_new)
    u = jnp.exp(s - m_new)
    l_blk = jnp.sum(u, axis=-1, keepdims=True)
    l_old = l_ref[...]
    ecorr = jnp.exp(corr)
    l_new = ecorr * l_old + l_blk
    acc = (ecorr * l_old) * o_ref[0]
    res = acc + _dot1(u, v_ref[0])
    o_ref[0] = res * (1.0 / l_new)
    m_ref[...] = m_new
    l_ref[...] = l_new


def _proj_body(ctx_ref, wao_ref, bao_ref, x_ref, g_ref, b_ref, wr_ref, br_ref,
               ao_ref, sc_ref, eid_ref):
    t = _dot1(ctx_ref[...], wao_ref[...]) + bao_ref[...] + x_ref[...]
    mu = jnp.mean(t, axis=-1, keepdims=True)
    var = jnp.mean((t - mu) ** 2, axis=-1, keepdims=True)
    ao = (t - mu) / jnp.sqrt(var + 1e-12) * g_ref[...] + b_ref[...]
    ao_ref[...] = ao
    logits = _dot1(ao, wr_ref[...]) + br_ref[...]
    lane = jax.lax.broadcasted_iota(jnp.int32, logits.shape, 1)
    logits = jnp.where(lane < E, logits, jnp.float32(-1e30))
    m = jnp.max(logits, axis=-1, keepdims=True)
    p = jnp.exp(logits - m)
    p = p / jnp.sum(p, axis=-1, keepdims=True)
    sc_ref[...] = p[:, :E]
    mp = jnp.max(p, axis=-1, keepdims=True)
    eid_ref[...] = jnp.min(jnp.where(p == mp, lane, E), axis=-1,
                           keepdims=True)


def _moe_body(te_ref, act_ref, x_ref, we_ref, be_ref, wo_ref, y_ref):
    t = pl.program_id(0)
    c = pl.program_id(1)

    @pl.when(act_ref[t] != 0)
    def _():
        xb = x_ref[...].astype(BF16)
        we = we_ref[0].astype(BF16)
        h = jax.lax.dot_general(xb, we, (((1,), (1,)), ((), ())),
                                preferred_element_type=F32)
        h = h + be_ref[0]
        h = 0.5 * h * (1.0 + jax.lax.erf(h * 0.7071067811865476))
        wo = wo_ref[:, pl.ds(c * DFFC, DFFC)]
        y = jax.lax.dot_general(h.astype(BF16), wo,
                                (((1,), (1,)), ((), ())),
                                preferred_element_type=F32)

        @pl.when(c == 0)
        def _():
            y_ref[...] = y

        @pl.when(c != 0)
        def _():
            y_ref[...] += y


NSUB = 16
CHUNK = S // NSUB
HALF = CHUNK // 2
NGRP = CHUNK // 16
LOG2T = 8

_sc_mesh = plsc.VectorSubcoreMesh(core_axis_name="c", subcore_axis_name="s")


@functools.partial(
    pl.kernel,
    mesh=_sc_mesh,
    out_type=[
        jax.ShapeDtypeStruct((MAXP, D), jnp.float32),
        jax.ShapeDtypeStruct((S,), jnp.int32),
        jax.ShapeDtypeStruct((MAX_TILES,), jnp.int32),
        jax.ShapeDtypeStruct((MAX_TILES,), jnp.int32),
    ],
    scratch_types=[
        pltpu.VMEM((S,), jnp.int32),
        pltpu.VMEM((2, HALF), jnp.int32),
        pltpu.VMEM((HALF, D), jnp.float32),
        pltpu.VMEM((16,), jnp.int32),
        pltpu.SemaphoreType.DMA,
    ],
)
def _sc_dispatch(eids_hbm, ao_hbm, x_pad_hbm, pos_hbm, te_hbm, act_hbm,
                 eids_v, pos_v, rows_v, meta_v, sem):
    c = lax.axis_index("c")
    s = lax.axis_index("s")
    lane = lax.iota(jnp.int32, 16)
    one = jnp.full((16,), 1, jnp.int32)
    zero = jnp.zeros((16,), jnp.int32)

    def take16(x, idx):
        return x.at[idx].get(mode="promise_in_bounds")

    def bcast(x, j):
        return take16(x, jnp.full((16,), j, jnp.int32))

    def eq_mask(a, b):
        return one - jnp.minimum(jnp.abs(a - b), one)

    def ge_mask(a, b):
        return jnp.minimum(jnp.maximum(a - b + 1, zero), one)

    def sum16(x):
        for kk in (1, 2, 4, 8):
            x = x + take16(x, (lane + kk) & 15)
        return x

    def cumsum16(x):
        for kk in (1, 2, 4, 8):
            sh = take16(x, jnp.maximum(lane - kk, 0))
            x = x + ge_mask(lane, jnp.full((16,), kk, jnp.int32)) * sh
        return x

    pltpu.sync_copy(eids_hbm, eids_v)
    e_consts = [jnp.full((16,), e, jnp.int32) for e in range(E)]
    lane_e = [eq_mask(lane, e_consts[e]) for e in range(E)]
    sgrp = jnp.broadcast_to(s * NGRP, (16,)).astype(jnp.int32)
    acc_t = [zero for _ in range(E)]
    acc_p = [zero for _ in range(E)]
    for gi in range(NSUB * NGRP):
        ev = eids_v[pl.ds(gi * 16, 16)]
        infl = ge_mask(sgrp, jnp.full((16,), gi + 1, jnp.int32))
        for e in range(E):
            m = eq_mask(ev, e_consts[e])
            acc_t[e] = acc_t[e] + m
            acc_p[e] = acc_p[e] + infl * m
    tot = zero
    prefix = zero
    for e in range(E):
        tot = tot + lane_e[e] * sum16(acc_t[e])
        prefix = prefix + lane_e[e] * sum16(acc_p[e])
    padded = ((tot + (T - 1)) >> LOG2T) << LOG2T
    incl = cumsum16(padded)
    base = incl - padded
    ntiles_v = bcast(incl, 15) >> LOG2T
    off = base + prefix

    for g in range(NGRP):
        ev = eids_v[pl.ds(s * CHUNK + g * 16, 16)]
        pv = jnp.zeros((16,), jnp.int32)
        for e in range(E):
            e_vec = jnp.full((16,), e, jnp.int32)
            m = eq_mask(ev, e_vec)
            csum = cumsum16(m)
            rank = csum - 1
            off_e = bcast(off, e)
            pv = m * (off_e + rank) + (one - m) * pv
            off = off + eq_mask(lane, e_vec) * bcast(csum, 15)
        pos_v[g // (NGRP // 2), pl.ds((g % (NGRP // 2)) * 16, 16)] = pv

    pltpu.sync_copy(pos_v.at[c], pos_hbm.at[pl.ds(s * CHUNK + c * HALF, HALF)])
    pltpu.sync_copy(ao_hbm.at[pl.ds(s * CHUNK + c * HALF, HALF)], rows_v)
    pltpu.async_copy(rows_v, x_pad_hbm.at[pos_v.at[c]], sem).wait()

    @pl.when(jnp.logical_and(c == 0, s == 0))
    def _():
        te = jnp.zeros((16,), jnp.int32)
        endt = (base + padded) >> LOG2T
        for e in range(E):
            te = te + ge_mask(lane, bcast(endt, e))
        te = jnp.minimum(te, E - 1)
        last_e = take16(te, jnp.maximum(ntiles_v - 1, zero))
        act = one - ge_mask(lane, ntiles_v)
        te = act * te + (one - act) * last_e
        meta_v[...] = te
        pltpu.sync_copy(meta_v, te_hbm)
        meta_v[...] = act
        pltpu.sync_copy(meta_v, act_hbm)


@functools.partial(
    pl.kernel,
    mesh=_sc_mesh,
    out_type=jax.ShapeDtypeStruct((S, D), jnp.float32),
    scratch_types=[
        pltpu.VMEM((2, HALF), jnp.int32),
        pltpu.VMEM((HALF, D), jnp.float32),
        pltpu.SemaphoreType.DMA,
    ],
)
def _sc_unsort(pos_hbm, y_pad_hbm, y_hbm, pos_v, rows_v, sem):
    c = lax.axis_index("c")
    s = lax.axis_index("s")
    base = s * CHUNK + c * HALF
    pltpu.sync_copy(pos_hbm.at[pl.ds(base, HALF)], pos_v.at[c])
    pltpu.async_copy(y_pad_hbm.at[pos_v.at[c]], rows_v, sem).wait()
    pltpu.sync_copy(rows_v, y_hbm.at[pl.ds(base, HALF)])


def _ln2_body(y_ref, bo_ref, ao_ref, g_ref, b_ref, o_ref):
    tot = y_ref[...] + bo_ref[...] + ao_ref[...]
    mu = jnp.mean(tot, axis=-1, keepdims=True)
    var = jnp.mean((tot - mu) ** 2, axis=-1, keepdims=True)
    o_ref[...] = (tot - mu) / jnp.sqrt(var + 1e-12) * g_ref[...] + b_ref[...]


def kernel(hidden_states, Wq, bq, Wk, bk, Wv, bv, Wao, bao, ln1_g, ln1_b,
           Wr, br, We, be, Wo, bo, ln2_g, ln2_b):
    x = hidden_states.reshape(S, D)
    wqkv = jnp.concatenate([Wq.T, Wk.T, Wv.T], axis=1)
    bqkv = jnp.concatenate([bq, bk, bv]).reshape(1, 3 * D)

    qkv = pl.pallas_call(
        _qkv_body,
        grid=(S // TR,),
        in_specs=[pl.BlockSpec((TR, D), lambda i: (i, 0)),
                  pl.BlockSpec((D, 3 * D), lambda i: (0, 0)),
                  pl.BlockSpec((1, 3 * D), lambda i: (0, 0))],
        out_specs=pl.BlockSpec((TR, 3 * D), lambda i: (i, 0)),
        out_shape=jax.ShapeDtypeStruct((S, 3 * D), F32),
    )(x, wqkv, bqkv)

    q = qkv[:, :D].reshape(S, H, DH).transpose(1, 0, 2)
    k = qkv[:, D:2 * D].reshape(S, H, DH).transpose(1, 0, 2)
    v = qkv[:, 2 * D:].reshape(S, H, DH).transpose(1, 0, 2)

    ctx3 = pl.pallas_call(
        _attn_body,
        grid=(H, S // TQ, S // TKB),
        in_specs=[pl.BlockSpec((1, TQ, DH), lambda h, i, j: (h, i, 0)),
                  pl.BlockSpec((1, TKB, DH), lambda h, i, j: (h, j, 0)),
                  pl.BlockSpec((1, TKB, DH), lambda h, i, j: (h, j, 0))],
        out_specs=pl.BlockSpec((1, TQ, DH), lambda h, i, j: (h, i, 0)),
        out_shape=jax.ShapeDtypeStruct((H, S, DH), F32),
        scratch_shapes=[pltpu.VMEM((TQ, 1), F32),
                        pltpu.VMEM((TQ, 1), F32)],
    )(q, k, v)
    ctx = ctx3.transpose(1, 0, 2).reshape(S, D)

    wr_pad = jnp.zeros((D, 128), F32).at[:, :E].set(Wr.T)
    br_pad = jnp.zeros((1, 128), F32).at[0, :E].set(br)
    ao, scores, eids = pl.pallas_call(
        _proj_body,
        grid=(S // TR,),
        in_specs=[pl.BlockSpec((TR, D), lambda i: (i, 0)),
                  pl.BlockSpec((D, D), lambda i: (0, 0)),
                  pl.BlockSpec((1, D), lambda i: (0, 0)),
                  pl.BlockSpec((TR, D), lambda i: (i, 0)),
                  pl.BlockSpec((1, D), lambda i: (0, 0)),
                  pl.BlockSpec((1, D), lambda i: (0, 0)),
                  pl.BlockSpec((D, 128), lambda i: (0, 0)),
                  pl.BlockSpec((1, 128), lambda i: (0, 0))],
        out_specs=[pl.BlockSpec((TR, D), lambda i: (i, 0)),
                   pl.BlockSpec((TR, E), lambda i: (i, 0)),
                   pl.BlockSpec((TR, 1), lambda i: (i, 0))],
        out_shape=[jax.ShapeDtypeStruct((S, D), F32),
                   jax.ShapeDtypeStruct((S, E), F32),
                   jax.ShapeDtypeStruct((S, 1), jnp.int32)],
    )(ctx, Wao.T, bao.reshape(1, D), x, ln1_g.reshape(1, D),
      ln1_b.reshape(1, D), wr_pad, br_pad)

    x_pad, pos, te, act = _sc_dispatch(eids.reshape(S), ao)

    grid_spec = pltpu.PrefetchScalarGridSpec(
        num_scalar_prefetch=2,
        grid=(MAX_TILES, NDC),
        in_specs=[pl.BlockSpec((T, D), lambda t, c, te, a: (t, 0)),
                  pl.BlockSpec((1, DFFC, D), lambda t, c, te, a: (te[t], c, 0)),
                  pl.BlockSpec((1, 1, DFFC),
                               lambda t, c, te, a: (te[t] * NDC + c, 0, 0)),
                  pl.BlockSpec((D, DFF), lambda t, c, te, a: (0, 0))],
        out_specs=pl.BlockSpec((T, D), lambda t, c, te, a: (t, 0)),
    )
    y_pad = pl.pallas_call(
        _moe_body,
        grid_spec=grid_spec,
        out_shape=jax.ShapeDtypeStruct((MAXP, D), F32),
    )(te, act, x_pad, We, be.reshape(E * NDC, 1, DFFC), Wo.astype(BF16), )

    y = _sc_unsort(pos, y_pad)

    out = pl.pallas_call(
        _ln2_body,
        grid=(S // TR,),
        in_specs=[pl.BlockSpec((TR, D), lambda i: (i, 0)),
                  pl.BlockSpec((1, D), lambda i: (0, 0)),
                  pl.BlockSpec((TR, D), lambda i: (i, 0)),
                  pl.BlockSpec((1, D), lambda i: (0, 0)),
                  pl.BlockSpec((1, D), lambda i: (0, 0))],
        out_specs=pl.BlockSpec((TR, D), lambda i: (i, 0)),
        out_shape=jax.ShapeDtypeStruct((S, D), F32),
    )(y, bo.reshape(1, D), ao, ln2_g.reshape(1, D), ln2_b.reshape(1, D))

    return (out.reshape(1, S, D), scores.reshape(1, S, E))


# speedup vs baseline: 2.0989x; 1.0499x over previous
import functools

import jax
import jax.numpy as jnp
from jax import lax
from jax.experimental import pallas as pl
from jax.experimental.pallas import tpu as pltpu
from jax.experimental.pallas import tpu_sc as plsc

S, D, H, DFF, E = 2048, 1024, 16, 4096, 8
DH = D // H
TQ = 1024
TKB = 1024
TR = 256
T = 256
MAXP = S + E * T
MAX_TILES = MAXP // T
DFFC = 4096
NDC = DFF // DFFC
F32 = jnp.float32
BF16 = jnp.bfloat16
HIGHEST = jax.lax.Precision.HIGHEST


_NN = (((1,), (0,)), ((), ()))
_NT = (((1,), (1,)), ((), ()))


def _hi_lo(x):
    h = x.astype(BF16)
    l = (x - h.astype(F32)).astype(BF16)
    return h, l


def _dot1(x, w, dn=_NN):
    return jax.lax.dot_general(x.astype(BF16), w.astype(BF16),
                               dimension_numbers=dn,
                               preferred_element_type=F32)


def _split3(x):
    h = x.astype(BF16)
    r = x - h.astype(F32)
    m = r.astype(BF16)
    l = (r - m.astype(F32)).astype(BF16)
    return h, m, l


def _dot6(x, w, dn=_NN):
    xh, xm, xl = _split3(x)
    wh, wm, wl = _split3(w)
    d = functools.partial(jax.lax.dot_general, dimension_numbers=dn,
                          preferred_element_type=F32)
    small = d(xh, wl) + d(xl, wh) + d(xm, wm)
    mid = d(xh, wm) + d(xm, wh)
    return (small + mid) + d(xh, wh)


def _qkv_body(x_ref, w_ref, b_ref, o_ref):
    o_ref[...] = _dot1(x_ref[...], w_ref[...]) + b_ref[...]


def _attn_body(q_ref, k_ref, v_ref, o_ref, m_ref, l_ref):
    kb = pl.program_id(2)

    @pl.when(kb == 0)
    def _():
        o_ref[0] = jnp.zeros_like(o_ref[0])
        m_ref[...] = jnp.full_like(m_ref[...], -jnp.inf)
        l_ref[...] = jnp.zeros_like(l_ref[...])

    s = _dot1(q_ref[0], k_ref[0], _NT) * 0.125
    m_blk = jnp.max(s, axis=-1, keepdims=True)
    m_old = m_ref[...]
    m_new = jnp.maximum(m_old, m_blk)
    corr = jnp.where(m_old == m_new, jnp.float32(0.0), m_old - m_new)
    u = jnp.exp(s - m_new)
    l_blk = jnp.sum(u, axis=-1, keepdims=True)
    l_old = l_ref[...]
    ecorr = jnp.exp(corr)
    l_new = ecorr * l_old + l_blk
    acc = (ecorr * l_old) * o_ref[0]
    res = acc + _dot1(u, v_ref[0])
    o_ref[0] = res * (1.0 / l_new)
    m_ref[...] = m_new
    l_ref[...] = l_new


def _proj_body(ctx_ref, wao_ref, bao_ref, x_ref, g_ref, b_ref, wr_ref, br_ref,
               ao_ref, sc_ref, eid_ref):
    t = _dot1(ctx_ref[...], wao_ref[...]) + bao_ref[...] + x_ref[...]
    mu = jnp.mean(t, axis=-1, keepdims=True)
    var = jnp.mean((t - mu) ** 2, axis=-1, keepdims=True)
    ao = (t - mu) / jnp.sqrt(var + 1e-12) * g_ref[...] + b_ref[...]
    ao_ref[...] = ao
    logits = _dot1(ao, wr_ref[...]) + br_ref[...]
    lane = jax.lax.broadcasted_iota(jnp.int32, logits.shape, 1)
    logits = jnp.where(lane < E, logits, jnp.float32(-1e30))
    m = jnp.max(logits, axis=-1, keepdims=True)
    p = jnp.exp(logits - m)
    p = p / jnp.sum(p, axis=-1, keepdims=True)
    sc_ref[...] = p[:, :E]
    mp = jnp.max(p, axis=-1, keepdims=True)
    eid_ref[...] = jnp.min(jnp.where(p == mp, lane, E), axis=-1,
                           keepdims=True)


def _moe_body(te_ref, act_ref, x_ref, we_ref, be_ref, wo_ref, y_ref):
    t = pl.program_id(0)
    c = pl.program_id(1)

    @pl.when(act_ref[t] != 0)
    def _():
        xb = x_ref[...].astype(BF16)
        we = we_ref[0].astype(BF16)
        h = jax.lax.dot_general(xb, we, (((1,), (1,)), ((), ())),
                                preferred_element_type=F32)
        h = h + be_ref[0]
        h = 0.5 * h * (1.0 + jax.lax.erf(h * 0.7071067811865476))
        wo = wo_ref[:, pl.ds(c * DFFC, DFFC)]
        y = jax.lax.dot_general(h.astype(BF16), wo,
                                (((1,), (1,)), ((), ())),
                                preferred_element_type=F32)

        @pl.when(c == 0)
        def _():
            y_ref[...] = y

        @pl.when(c != 0)
        def _():
            y_ref[...] += y


NSUB = 16
CHUNK = S // NSUB
HALF = CHUNK // 2
NGRP = CHUNK // 16
LOG2T = 8

_sc_mesh = plsc.VectorSubcoreMesh(core_axis_name="c", subcore_axis_name="s")


@functools.partial(
    pl.kernel,
    mesh=_sc_mesh,
    out_type=[
        jax.ShapeDtypeStruct((MAXP, D), jnp.float32),
        jax.ShapeDtypeStruct((S,), jnp.int32),
        jax.ShapeDtypeStruct((MAX_TILES,), jnp.int32),
        jax.ShapeDtypeStruct((MAX_TILES,), jnp.int32),
    ],
    scratch_types=[
        pltpu.VMEM((S,), jnp.int32),
        pltpu.VMEM((2, HALF), jnp.int32),
        pltpu.VMEM((HALF, D), jnp.float32),
        pltpu.VMEM((16,), jnp.int32),
        pltpu.SemaphoreType.DMA,
    ],
)
def _sc_dispatch(eids_hbm, ao_hbm, x_pad_hbm, pos_hbm, te_hbm, act_hbm,
                 eids_v, pos_v, rows_v, meta_v, sem):
    c = lax.axis_index("c")
    s = lax.axis_index("s")
    lane = lax.iota(jnp.int32, 16)
    one = jnp.full((16,), 1, jnp.int32)
    zero = jnp.zeros((16,), jnp.int32)

    def take16(x, idx):
        return x.at[idx].get(mode="promise_in_bounds")

    def bcast(x, j):
        return take16(x, jnp.full((16,), j, jnp.int32))

    def eq_mask(a, b):
        return one - jnp.minimum(jnp.abs(a - b), one)

    def ge_mask(a, b):
        return jnp.minimum(jnp.maximum(a - b + 1, zero), one)

    def sum16(x):
        for kk in (1, 2, 4, 8):
            x = x + take16(x, (lane + kk) & 15)
        return x

    def cumsum16(x):
        for kk in (1, 2, 4, 8):
            sh = take16(x, jnp.maximum(lane - kk, 0))
            x = x + ge_mask(lane, jnp.full((16,), kk, jnp.int32)) * sh
        return x

    pltpu.sync_copy(eids_hbm, eids_v)
    e_consts = [jnp.full((16,), e, jnp.int32) for e in range(E)]
    lane_e = [eq_mask(lane, e_consts[e]) for e in range(E)]
    sgrp = jnp.broadcast_to(s * NGRP, (16,)).astype(jnp.int32)
    acc_t = [zero for _ in range(E)]
    acc_p = [zero for _ in range(E)]
    for gi in range(NSUB * NGRP):
        ev = eids_v[pl.ds(gi * 16, 16)]
        infl = ge_mask(sgrp, jnp.full((16,), gi + 1, jnp.int32))
        for e in range(E):
            m = eq_mask(ev, e_consts[e])
            acc_t[e] = acc_t[e] + m
            acc_p[e] = acc_p[e] + infl * m
    tot = zero
    prefix = zero
    for e in range(E):
        tot = tot + lane_e[e] * sum16(acc_t[e])
        prefix = prefix + lane_e[e] * sum16(acc_p[e])
    padded = ((tot + (T - 1)) >> LOG2T) << LOG2T
    incl = cumsum16(padded)
    base = incl - padded
    ntiles_v = bcast(incl, 15) >> LOG2T
    off = base + prefix

    for g in range(NGRP):
        ev = eids_v[pl.ds(s * CHUNK + g * 16, 16)]
        pv = jnp.zeros((16,), jnp.int32)
        for e in range(E):
            e_vec = jnp.full((16,), e, jnp.int32)
            m = eq_mask(ev, e_vec)
            csum = cumsum16(m)
            rank = csum - 1
            off_e = bcast(off, e)
            pv = m * (off_e + rank) + (one - m) * pv
            off = off + eq_mask(lane, e_vec) * bcast(csum, 15)
        pos_v[g // (NGRP // 2), pl.ds((g % (NGRP // 2)) * 16, 16)] = pv

    pltpu.sync_copy(pos_v.at[c], pos_hbm.at[pl.ds(s * CHUNK + c * HALF, HALF)])
    pltpu.sync_copy(ao_hbm.at[pl.ds(s * CHUNK + c * HALF, HALF)], rows_v)
    pltpu.async_copy(rows_v, x_pad_hbm.at[pos_v.at[c]], sem).wait()

    @pl.when(jnp.logical_and(c == 0, s == 0))
    def _():
        te = jnp.zeros((16,), jnp.int32)
        endt = (base + padded) >> LOG2T
        for e in range(E):
            te = te + ge_mask(lane, bcast(endt, e))
        te = jnp.minimum(te, E - 1)
        last_e = take16(te, jnp.maximum(ntiles_v - 1, zero))
        act = one - ge_mask(lane, ntiles_v)
        te = act * te + (one - act) * last_e
        meta_v[...] = te
        pltpu.sync_copy(meta_v, te_hbm)
        meta_v[...] = act
        pltpu.sync_copy(meta_v, act_hbm)


@functools.partial(
    pl.kernel,
    mesh=_sc_mesh,
    out_type=jax.ShapeDtypeStruct((S, D), jnp.float32),
    scratch_types=[
        pltpu.VMEM((2, HALF), jnp.int32),
        pltpu.VMEM((HALF, D), jnp.float32),
        pltpu.SemaphoreType.DMA,
    ],
)
def _sc_unsort(pos_hbm, y_pad_hbm, y_hbm, pos_v, rows_v, sem):
    c = lax.axis_index("c")
    s = lax.axis_index("s")
    base = s * CHUNK + c * HALF
    pltpu.sync_copy(pos_hbm.at[pl.ds(base, HALF)], pos_v.at[c])
    pltpu.async_copy(y_pad_hbm.at[pos_v.at[c]], rows_v, sem).wait()
    pltpu.sync_copy(rows_v, y_hbm.at[pl.ds(base, HALF)])


def _ln2_body(y_ref, bo_ref, ao_ref, g_ref, b_ref, o_ref):
    tot = y_ref[...] + bo_ref[...] + ao_ref[...]
    mu = jnp.mean(tot, axis=-1, keepdims=True)
    var = jnp.mean((tot - mu) ** 2, axis=-1, keepdims=True)
    o_ref[...] = (tot - mu) / jnp.sqrt(var + 1e-12) * g_ref[...] + b_ref[...]


def kernel(hidden_states, Wq, bq, Wk, bk, Wv, bv, Wao, bao, ln1_g, ln1_b,
           Wr, br, We, be, Wo, bo, ln2_g, ln2_b):
    x = hidden_states.reshape(S, D)
    wqkv = jnp.concatenate([Wq.T, Wk.T, Wv.T], axis=1)
    bqkv = jnp.concatenate([bq, bk, bv]).reshape(1, 3 * D)

    qkv = pl.pallas_call(
        _qkv_body,
        grid=(S // TR,),
        in_specs=[pl.BlockSpec((TR, D), lambda i: (i, 0)),
                  pl.BlockSpec((D, 3 * D), lambda i: (0, 0)),
                  pl.BlockSpec((1, 3 * D), lambda i: (0, 0))],
        out_specs=pl.BlockSpec((TR, 3 * D), lambda i: (i, 0)),
        out_shape=jax.ShapeDtypeStruct((S, 3 * D), F32),
    )(x, wqkv, bqkv)

    q = qkv[:, :D].reshape(S, H, DH).transpose(1, 0, 2)
    k = qkv[:, D:2 * D].reshape(S, H, DH).transpose(1, 0, 2)
    v = qkv[:, 2 * D:].reshape(S, H, DH).transpose(1, 0, 2)

    ctx3 = pl.pallas_call(
        _attn_body,
        grid=(H, S // TQ, S // TKB),
        in_specs=[pl.BlockSpec((1, TQ, DH), lambda h, i, j: (h, i, 0)),
                  pl.BlockSpec((1, TKB, DH), lambda h, i, j: (h, j, 0)),
                  pl.BlockSpec((1, TKB, DH), lambda h, i, j: (h, j, 0))],
        out_specs=pl.BlockSpec((1, TQ, DH), lambda h, i, j: (h, i, 0)),
        out_shape=jax.ShapeDtypeStruct((H, S, DH), F32),
        scratch_shapes=[pltpu.VMEM((TQ, 1), F32),
                        pltpu.VMEM((TQ, 1), F32)],
    )(q, k, v)
    ctx = ctx3.transpose(1, 0, 2).reshape(S, D)

    wr_pad = jnp.zeros((D, 128), F32).at[:, :E].set(Wr.T)
    br_pad = jnp.zeros((1, 128), F32).at[0, :E].set(br)
    ao, scores, eids = pl.pallas_call(
        _proj_body,
        grid=(S // TR,),
        in_specs=[pl.BlockSpec((TR, D), lambda i: (i, 0)),
                  pl.BlockSpec((D, D), lambda i: (0, 0)),
                  pl.BlockSpec((1, D), lambda i: (0, 0)),
                  pl.BlockSpec((TR, D), lambda i: (i, 0)),
                  pl.BlockSpec((1, D), lambda i: (0, 0)),
                  pl.BlockSpec((1, D), lambda i: (0, 0)),
                  pl.BlockSpec((D, 128), lambda i: (0, 0)),
                  pl.BlockSpec((1, 128), lambda i: (0, 0))],
        out_specs=[pl.BlockSpec((TR, D), lambda i: (i, 0)),
                   pl.BlockSpec((TR, E), lambda i: (i, 0)),
                   pl.BlockSpec((TR, 1), lambda i: (i, 0))],
        out_shape=[jax.ShapeDtypeStruct((S, D), F32),
                   jax.ShapeDtypeStruct((S, E), F32),
                   jax.ShapeDtypeStruct((S, 1), jnp.int32)],
    )(ctx, Wao.T, bao.reshape(1, D), x, ln1_g.reshape(1, D),
      ln1_b.reshape(1, D), wr_pad, br_pad)

    x_pad, pos, te, act = _sc_dispatch(eids.reshape(S), ao)

    grid_spec = pltpu.PrefetchScalarGridSpec(
        num_scalar_prefetch=2,
        grid=(MAX_TILES, NDC),
        in_specs=[pl.BlockSpec((T, D), lambda t, c, te, a: (t, 0)),
                  pl.BlockSpec((1, DFFC, D), lambda t, c, te, a: (te[t], c, 0)),
                  pl.BlockSpec((1, 1, DFFC),
                               lambda t, c, te, a: (te[t] * NDC + c, 0, 0)),
                  pl.BlockSpec((D, DFF), lambda t, c, te, a: (0, 0))],
        out_specs=pl.BlockSpec((T, D), lambda t, c, te, a: (t, 0)),
    )
    y_pad = pl.pallas_call(
        _moe_body,
        grid_spec=grid_spec,
        out_shape=jax.ShapeDtypeStruct((MAXP, D), F32),
    )(te, act, x_pad, We, be.reshape(E * NDC, 1, DFFC), Wo.astype(BF16), )

    y = _sc_unsort(pos, y_pad)

    out = pl.pallas_call(
        _ln2_body,
        grid=(S // TR,),
        in_specs=[pl.BlockSpec((TR, D), lambda i: (i, 0)),
                  pl.BlockSpec((1, D), lambda i: (0, 0)),
                  pl.BlockSpec((TR, D), lambda i: (i, 0)),
                  pl.BlockSpec((1, D), lambda i: (0, 0)),
                  pl.BlockSpec((1, D), lambda i: (0, 0))],
        out_specs=pl.BlockSpec((TR, D), lambda i: (i, 0)),
        out_shape=jax.ShapeDtypeStruct((S, D), F32),
    )(y, bo.reshape(1, D), ao, ln2_g.reshape(1, D), ln2_b.reshape(1, D))

    return (out.reshape(1, S, D), scores.reshape(1, S, E))
